# Initial kernel scaffold; baseline (speedup 1.0000x reference)
import jax
import jax.numpy as jnp
from jax.experimental import pallas as pl


def kernel(z, codebook):
    raise NotImplementedError("write your pallas kernel here")


# speedup vs baseline: 3.0221x; 3.0221x over previous
import functools

import jax
import jax.numpy as jnp
from jax.experimental import pallas as pl


def _vq_kernel(ninv, z_ref, cb_ref, zq_ref, idx_ref, loss_ref):
    b = pl.program_id(0)
    t = pl.program_id(1)

    zb = z_ref[0]
    cb = cb_ref[...]

    cnorm = jnp.sum(cb * cb, axis=1, keepdims=True)
    znorm = jnp.sum(zb * zb, axis=0, keepdims=True)
    mm = jax.lax.dot(cb, zb, preferred_element_type=jnp.float32)
    scores = (znorm + cnorm) - 2.0 * mm

    n = scores.shape[0]
    minval = jnp.min(scores, axis=0)
    row_iota = jax.lax.broadcasted_iota(jnp.int32, scores.shape, 0)
    idx = jnp.min(jnp.where(scores == minval[None, :], row_iota, n),
                  axis=0).astype(jnp.int32)
    idx_ref[0, 0, :] = idx

    onehot = (row_iota == idx[None, :]).astype(jnp.float32)
    zqb = jax.lax.dot_general(
        cb, onehot, (((0,), (0,)), ((), ())),
        preferred_element_type=jnp.float32)
    zq_ref[0] = zb + (zqb - zb)

    d = zqb - zb
    part = jnp.sum(d * d, keepdims=True).reshape(1, 1) * ninv

    @pl.when(jnp.logical_and(b == 0, t == 0))
    def _init():
        loss_ref[...] = jnp.zeros_like(part)

    loss_ref[...] += part


def kernel(z, codebook):
    B, C, T = z.shape
    N, _ = codebook.shape
    TT = 512
    grid = (B, T // TT)

    zq, idx3, loss = pl.pallas_call(
        functools.partial(_vq_kernel, 1.0 / float(z.size)),
        grid=grid,
        in_specs=[
            pl.BlockSpec((1, C, TT), lambda b, t: (b, 0, t)),
            pl.BlockSpec((N, C), lambda b, t: (0, 0)),
        ],
        out_specs=[
            pl.BlockSpec((1, C, TT), lambda b, t: (b, 0, t)),
            pl.BlockSpec((1, 1, TT), lambda b, t: (b, 0, t)),
            pl.BlockSpec((1, 1), lambda b, t: (0, 0)),
        ],
        out_shape=[
            jax.ShapeDtypeStruct((B, C, T), jnp.float32),
            jax.ShapeDtypeStruct((B, 1, T), jnp.int32),
            jax.ShapeDtypeStruct((1, 1), jnp.float32),
        ],
    )(z, codebook)

    return zq, idx3.reshape(B, T), loss[0, 0]
